# Initial kernel scaffold; baseline (speedup 1.0000x reference)
import jax
import jax.numpy as jnp
from jax.experimental import pallas as pl


def kernel(x, Wq, bq, Wk, bk, Wv, bv, part_emb, Wo, bo):
    raise NotImplementedError("write your pallas kernel here")


# speedup vs baseline: 11.9654x; 11.9654x over previous
import functools

import jax
import jax.numpy as jnp
from jax import lax
from jax.experimental import pallas as pl

B = 1
S = 2048
D = 768
H = 12
HD = D // H
P = 32
K = 2
R = 16
PR = P * R
NEG = -1e30


def _sse_head_kernel(x_ref, wq_ref, bq_ref, wk_ref, bk_ref, wv_ref, bv_ref,
                     pe_ref, wo_ref, bo_ref, y_ref):
    h = pl.program_id(0)
    xh = x_ref[:, 0, 0, :]
    wq = wq_ref[0]
    wk = wk_ref[0]
    wv = wv_ref[0]
    pe = pe_ref[0]

    q = jnp.dot(xh, wq, preferred_element_type=jnp.float32) + bq_ref[0]
    k = jnp.dot(xh, wk, preferred_element_type=jnp.float32) + bk_ref[0]
    v = jnp.dot(xh, wv, preferred_element_type=jnp.float32) + bv_ref[0]

    logits = jax.lax.dot_general(q, pe, (((1,), (1,)), ((), ())),
                                 preferred_element_type=jnp.float32)
    iota_p = lax.broadcasted_iota(jnp.int32, (S, P), 1)
    v1 = jnp.max(logits, axis=1, keepdims=True)
    i1 = jnp.min(jnp.where(logits == v1, iota_p, P), axis=1, keepdims=True)
    l2 = jnp.where(iota_p == i1, NEG, logits)
    v2 = jnp.max(l2, axis=1, keepdims=True)
    i2 = jnp.min(jnp.where(l2 == v2, iota_p, P), axis=1, keepdims=True)
    e = jnp.exp(v2 - v1)
    g1 = 1.0 / (1.0 + e)
    g2 = e / (1.0 + e)

    col_p = lax.broadcasted_iota(jnp.int32, (S, PR), 1) // R
    col_r = lax.broadcasted_iota(jnp.int32, (S, PR), 1) % R
    s_row = lax.broadcasted_iota(jnp.int32, (S, PR), 0) % R
    row_hit = col_r == s_row
    sel1 = col_p == i1
    sel2 = col_p == i2
    wfull = (jnp.where(sel1 & row_hit, g1, 0.0) +
             jnp.where(sel2 & row_hit, g2, 0.0))

    st_k = jax.lax.dot_general(wfull, k, (((0,), (0,)), ((), ())),
                               preferred_element_type=jnp.float32)
    st_v = jax.lax.dot_general(wfull, v, (((0,), (0,)), ((), ())),
                               preferred_element_type=jnp.float32)

    scores = jax.lax.dot_general(q, st_k, (((1,), (1,)), ((), ())),
                                 preferred_element_type=jnp.float32)
    scores = scores * (1.0 / 8.0)
    masked = jnp.where(sel1 | sel2, scores, NEG)
    m = jnp.max(masked, axis=1, keepdims=True)
    ex = jnp.exp(masked - m)
    attn = ex / jnp.sum(ex, axis=1, keepdims=True)

    out_h = jnp.dot(attn, st_v, preferred_element_type=jnp.float32)

    contrib = jax.lax.dot_general(out_h, wo_ref[:, 0, 0, :],
                                  (((1,), (1,)), ((), ())),
                                  preferred_element_type=jnp.float32)

    @pl.when(h == 0)
    def _():
        y_ref[...] = contrib + bo_ref[...]

    @pl.when(h > 0)
    def _():
        y_ref[...] += contrib


@functools.partial(jax.jit, static_argnames=("interpret",))
def _sse_call(x2d, Wq, bq, Wk, bk, Wv, bv, part_emb, Wo, bo2d, interpret=False):
    grid = (H,)
    out = pl.pallas_call(
        _sse_head_kernel,
        grid=grid,
        in_specs=[
            pl.BlockSpec((S, 1, 1, HD), lambda h: (0, h, 0, 0)),
            pl.BlockSpec((1, HD, HD), lambda h: (h, 0, 0)),
            pl.BlockSpec((1, 1, HD), lambda h: (h, 0, 0)),
            pl.BlockSpec((1, HD, HD), lambda h: (h, 0, 0)),
            pl.BlockSpec((1, 1, HD), lambda h: (h, 0, 0)),
            pl.BlockSpec((1, HD, HD), lambda h: (h, 0, 0)),
            pl.BlockSpec((1, 1, HD), lambda h: (h, 0, 0)),
            pl.BlockSpec((1, P, HD), lambda h: (h, 0, 0)),
            pl.BlockSpec((D, 1, 1, HD), lambda h: (0, h, 0, 0)),
            pl.BlockSpec((1, D), lambda h: (0, 0)),
        ],
        out_specs=pl.BlockSpec((S, D), lambda h: (0, 0)),
        out_shape=jax.ShapeDtypeStruct((S, D), jnp.float32),
        interpret=interpret,
    )(x2d, Wq, bq, Wk, bk, Wv, bv, part_emb, Wo, bo2d)
    return out


def kernel(x, Wq, bq, Wk, bk, Wv, bv, part_emb, Wo, bo, interpret=False):
    x4d = x.reshape(S, H, 1, HD)
    wo4d = Wo.reshape(D, H, 1, HD)
    y = _sse_call(x4d, Wq, bq.reshape(H, 1, HD), Wk, bk.reshape(H, 1, HD),
                  Wv, bv.reshape(H, 1, HD), part_emb, wo4d,
                  bo.reshape(1, D), interpret=interpret)
    return y.reshape(B, S, D)


# speedup vs baseline: 15.8300x; 1.3230x over previous
import functools

import jax
import jax.numpy as jnp
from jax import lax
from jax.experimental import pallas as pl
from jax.experimental.pallas import tpu as pltpu

B = 1
S = 2048
D = 768
H = 12
HD = D // H
P = 32
K = 2
R = 16
PR = P * R
NEG = -1e30


def _sse_head_kernel(x_ref, wq_ref, bq_ref, wk_ref, bk_ref, wv_ref, bv_ref,
                     pe_ref, e2_ref, rowmask_ref, wo_ref, bo_ref, y_ref,
                     concat_ref):
    h = pl.program_id(0)
    xh = x_ref[:, 0, 0, :]
    wq = wq_ref[0]
    wk = wk_ref[0]
    wv = wv_ref[0]
    pe = pe_ref[0]

    q = jnp.dot(xh, wq, preferred_element_type=jnp.float32) + bq_ref[0]
    k = jnp.dot(xh, wk, preferred_element_type=jnp.float32) + bk_ref[0]
    v = jnp.dot(xh, wv, preferred_element_type=jnp.float32) + bv_ref[0]

    logits = jax.lax.dot_general(pe, q, (((1,), (1,)), ((), ())),
                                 preferred_element_type=jnp.float32)
    iota_p = lax.broadcasted_iota(jnp.int32, (P, S), 0)
    v1 = jnp.max(logits, axis=0, keepdims=True)
    i1 = jnp.min(jnp.where(logits == v1, iota_p, P), axis=0, keepdims=True)
    hit1 = iota_p == i1
    l2 = jnp.where(hit1, NEG, logits)
    v2 = jnp.max(l2, axis=0, keepdims=True)
    i2 = jnp.min(jnp.where(l2 == v2, iota_p, P), axis=0, keepdims=True)
    hit2 = iota_p == i2
    e = jnp.exp(v2 - v1)
    g1 = 1.0 / (1.0 + e)
    g2 = e / (1.0 + e)

    w_sp = jnp.where(hit1, g1, 0.0) + jnp.where(hit2, g2, 0.0)
    addm = jnp.where(hit1 | hit2, 0.0, NEG)

    gate_b = jax.lax.dot_general(e2_ref[...], w_sp, (((1,), (0,)), ((), ())),
                                 preferred_element_type=jnp.float32)
    wfull = gate_b * rowmask_ref[...]

    st_k = jax.lax.dot_general(wfull, k, (((1,), (0,)), ((), ())),
                               preferred_element_type=jnp.float32)
    st_v = jax.lax.dot_general(wfull, v, (((1,), (0,)), ((), ())),
                               preferred_element_type=jnp.float32)

    q8 = q * (1.0 / 8.0)
    scores = jax.lax.dot_general(st_k, q8, (((1,), (1,)), ((), ())),
                                 preferred_element_type=jnp.float32)
    addmask = jax.lax.dot_general(e2_ref[...], addm, (((1,), (0,)), ((), ())),
                                  preferred_element_type=jnp.float32)
    masked = scores + addmask
    m = jnp.max(masked, axis=0, keepdims=True)
    ex = jnp.exp(masked - m)
    attn = ex * (1.0 / jnp.sum(ex, axis=0, keepdims=True))

    out_ht = jax.lax.dot_general(st_v, attn, (((0,), (0,)), ((), ())),
                                 preferred_element_type=jnp.float32)
    concat_ref[pl.ds(h * HD, HD), :] = out_ht

    @pl.when(h == H - 1)
    def _():
        y_ref[...] = jax.lax.dot_general(
            concat_ref[...], wo_ref[...], (((0,), (1,)), ((), ())),
            preferred_element_type=jnp.float32) + bo_ref[...]


@functools.partial(jax.jit, static_argnames=("interpret",))
def _sse_call(x4d, Wq, bq, Wk, bk, Wv, bv, part_emb, e2, rowmask, Wo, bo2d,
              interpret=False):
    grid = (H,)
    out = pl.pallas_call(
        _sse_head_kernel,
        grid=grid,
        in_specs=[
            pl.BlockSpec((S, 1, 1, HD), lambda h: (0, h, 0, 0)),
            pl.BlockSpec((1, HD, HD), lambda h: (h, 0, 0)),
            pl.BlockSpec((1, 1, HD), lambda h: (h, 0, 0)),
            pl.BlockSpec((1, HD, HD), lambda h: (h, 0, 0)),
            pl.BlockSpec((1, 1, HD), lambda h: (h, 0, 0)),
            pl.BlockSpec((1, HD, HD), lambda h: (h, 0, 0)),
            pl.BlockSpec((1, 1, HD), lambda h: (h, 0, 0)),
            pl.BlockSpec((1, P, HD), lambda h: (h, 0, 0)),
            pl.BlockSpec((PR, P), lambda h: (0, 0)),
            pl.BlockSpec((PR, S), lambda h: (0, 0)),
            pl.BlockSpec((D, D), lambda h: (0, 0)),
            pl.BlockSpec((1, D), lambda h: (0, 0)),
        ],
        out_specs=pl.BlockSpec((S, D), lambda h: (0, 0)),
        out_shape=jax.ShapeDtypeStruct((S, D), jnp.float32),
        scratch_shapes=[pltpu.VMEM((D, S), jnp.float32)],
        interpret=interpret,
    )(x4d, Wq, bq, Wk, bk, Wv, bv, part_emb, e2, rowmask, Wo, bo2d)
    return out


def kernel(x, Wq, bq, Wk, bk, Wv, bv, part_emb, Wo, bo, interpret=False):
    x4d = x.reshape(S, H, 1, HD)
    e2 = (jnp.arange(PR)[:, None] // R ==
          jnp.arange(P)[None, :]).astype(jnp.float32)
    rowmask = (jnp.arange(PR)[:, None] % R ==
               jnp.arange(S)[None, :] % R).astype(jnp.float32)
    y = _sse_call(x4d, Wq, bq.reshape(H, 1, HD), Wk, bk.reshape(H, 1, HD),
                  Wv, bv.reshape(H, 1, HD), part_emb, e2, rowmask, Wo,
                  bo.reshape(1, D), interpret=interpret)
    return y.reshape(B, S, D)


# speedup vs baseline: 16.6779x; 1.0536x over previous
import functools

import jax
import jax.numpy as jnp
from jax import lax
from jax.experimental import pallas as pl
from jax.experimental.pallas import tpu as pltpu

B = 1
S = 2048
D = 768
H = 12
HD = D // H
P = 32
K = 2
R = 16
M = S // R
PR = P * R
NEG = -1e30


def _sse_head_kernel(x_ref, wq_ref, bq_ref, wk_ref, bk_ref, wv_ref, bv_ref,
                     pe_ref, wo_ref, bo_ref, y_ref, concat_ref):
    h = pl.program_id(0)
    xh = x_ref[:, 0, 0, :]
    wq = wq_ref[0]
    wk = wk_ref[0]
    wv = wv_ref[0]
    pe = pe_ref[0]

    q = jnp.dot(xh, wq, preferred_element_type=jnp.float32) + bq_ref[0]
    k = jnp.dot(xh, wk, preferred_element_type=jnp.float32) + bk_ref[0]
    v = jnp.dot(xh, wv, preferred_element_type=jnp.float32) + bv_ref[0]

    logits = jax.lax.dot_general(pe, q, (((1,), (1,)), ((), ())),
                                 preferred_element_type=jnp.float32)
    iota_p = lax.broadcasted_iota(jnp.int32, (P, S), 0)
    v1 = jnp.max(logits, axis=0, keepdims=True)
    i1 = jnp.min(jnp.where(logits == v1, iota_p, P), axis=0, keepdims=True)
    hit1 = iota_p == i1
    l2 = jnp.where(hit1, NEG, logits)
    v2 = jnp.max(l2, axis=0, keepdims=True)
    i2 = jnp.min(jnp.where(l2 == v2, iota_p, P), axis=0, keepdims=True)
    hit2 = iota_p == i2
    e = jnp.exp(v2 - v1)
    g1 = 1.0 / (1.0 + e)
    g2 = e / (1.0 + e)

    w_sp = jnp.where(hit1, g1, 0.0) + jnp.where(hit2, g2, 0.0)
    addm = jnp.where(hit1 | hit2, 0.0, NEG)

    w4 = jnp.swapaxes(w_sp, 0, 1).reshape(M, R, P)
    k3 = k.reshape(M, R, HD)
    v3 = v.reshape(M, R, HD)
    dn = (((0,), (0,)), ((1,), (1,)))
    st_k = jax.lax.dot_general(w4, k3, dn,
                               preferred_element_type=jnp.float32).reshape(PR, HD)
    st_v = jax.lax.dot_general(w4, v3, dn,
                               preferred_element_type=jnp.float32).reshape(PR, HD)

    q8 = q * (1.0 / 8.0)
    scores = jax.lax.dot_general(st_k, q8, (((1,), (1,)), ((), ())),
                                 preferred_element_type=jnp.float32)
    addm_b = jnp.broadcast_to(addm[None, :, :], (R, P, S)).reshape(PR, S)
    masked = scores + addm_b
    m = jnp.max(masked, axis=0, keepdims=True)
    ex = jnp.exp(masked - m)
    attn = ex * (1.0 / jnp.sum(ex, axis=0, keepdims=True))

    out_ht = jax.lax.dot_general(st_v, attn, (((0,), (0,)), ((), ())),
                                 preferred_element_type=jnp.float32)
    concat_ref[pl.ds(h * HD, HD), :] = out_ht

    @pl.when(h == H - 1)
    def _():
        y_ref[...] = jax.lax.dot_general(
            concat_ref[...], wo_ref[...], (((0,), (1,)), ((), ())),
            preferred_element_type=jnp.float32) + bo_ref[...]


@functools.partial(jax.jit, static_argnames=("interpret",))
def _sse_call(x4d, Wq, bq, Wk, bk, Wv, bv, part_emb, Wo, bo2d,
              interpret=False):
    grid = (H,)
    out = pl.pallas_call(
        _sse_head_kernel,
        grid=grid,
        in_specs=[
            pl.BlockSpec((S, 1, 1, HD), lambda h: (0, h, 0, 0)),
            pl.BlockSpec((1, HD, HD), lambda h: (h, 0, 0)),
            pl.BlockSpec((1, 1, HD), lambda h: (h, 0, 0)),
            pl.BlockSpec((1, HD, HD), lambda h: (h, 0, 0)),
            pl.BlockSpec((1, 1, HD), lambda h: (h, 0, 0)),
            pl.BlockSpec((1, HD, HD), lambda h: (h, 0, 0)),
            pl.BlockSpec((1, 1, HD), lambda h: (h, 0, 0)),
            pl.BlockSpec((1, P, HD), lambda h: (h, 0, 0)),
            pl.BlockSpec((D, D), lambda h: (0, 0)),
            pl.BlockSpec((1, D), lambda h: (0, 0)),
        ],
        out_specs=pl.BlockSpec((S, D), lambda h: (0, 0)),
        out_shape=jax.ShapeDtypeStruct((S, D), jnp.float32),
        scratch_shapes=[pltpu.VMEM((D, S), jnp.float32)],
        interpret=interpret,
    )(x4d, Wq, bq, Wk, bk, Wv, bv, part_emb, Wo, bo2d)
    return out


def kernel(x, Wq, bq, Wk, bk, Wv, bv, part_emb, Wo, bo, interpret=False):
    x4d = x.reshape(S, H, 1, HD)
    y = _sse_call(x4d, Wq, bq.reshape(H, 1, HD), Wk, bk.reshape(H, 1, HD),
                  Wv, bv.reshape(H, 1, HD), part_emb, Wo,
                  bo.reshape(1, D), interpret=interpret)
    return y.reshape(B, S, D)
